# baseline (speedup 1.0000x reference)
import functools

import jax
import jax.numpy as jnp
from jax import lax
from jax.experimental import pallas as pl
from jax.experimental.pallas import tpu as pltpu
from jax.experimental.pallas import tpu_sc as plsc

N_EXP = 16
BLK = 1024
N_TOK = 8192
NW = 32
CHUNK = N_TOK // NW
TBLK = 16


def _matmul_kernel(w_ref, x_ref, b_ref, o_ref):
    comb = lax.dot_general(
        w_ref[...], x_ref[...],
        dimension_numbers=(((1,), (1,)), ((), ())),
        preferred_element_type=jnp.float32,
    )
    o_ref[...] = comb + b_ref[...]


def _route_kernel(comb_hbm, out_hbm, chunk_v, out_v):
    wid = lax.axis_index("s") * 2 + lax.axis_index("c")
    base = wid * CHUNK
    pltpu.sync_copy(comb_hbm.at[:, pl.ds(base, CHUNK)], chunk_v)

    def body(blk, carry):
        t0 = blk * TBLK
        m1 = jnp.full((TBLK,), -jnp.inf, jnp.float32)
        m2 = jnp.full((TBLK,), -jnp.inf, jnp.float32)
        e1 = jnp.zeros((TBLK,), jnp.float32)
        e2 = jnp.zeros((TBLK,), jnp.float32)
        for j in range(N_EXP):
            v = chunk_v[j, pl.ds(t0, TBLK)]
            ev = chunk_v[j + N_EXP, pl.ds(t0, TBLK)]
            gt1 = v > m1
            gt2 = v > m2
            e2 = jnp.where(gt1, e1, jnp.where(gt2, ev, e2))
            m2 = jnp.where(gt1, m1, jnp.where(gt2, v, m2))
            e1 = jnp.where(gt1, ev, e1)
            m1 = jnp.where(gt1, v, m1)
        w2 = jnp.exp(m2 - m1)
        out_v[pl.ds(t0, TBLK)] = (e1 + w2 * e2) / (1.0 + w2)
        return carry

    lax.fori_loop(0, CHUNK // TBLK, body, 0)
    pltpu.sync_copy(out_v, out_hbm.at[pl.ds(base, CHUNK)])


@functools.partial(jax.jit, static_argnames=())
def kernel(x, W_route, b_route, W_noise, b_noise, W_experts):
    n, d = x.shape
    wc = jnp.concatenate([W_noise, W_experts], axis=0)
    bias = jnp.concatenate([b_noise, jnp.zeros((N_EXP,), jnp.float32)])
    comb_t = pl.pallas_call(
        _matmul_kernel,
        grid=(n // BLK,),
        in_specs=[
            pl.BlockSpec((2 * N_EXP, d), lambda i: (0, 0)),
            pl.BlockSpec((BLK, d), lambda i: (i, 0)),
            pl.BlockSpec((2 * N_EXP, 1), lambda i: (0, 0)),
        ],
        out_specs=pl.BlockSpec((2 * N_EXP, BLK), lambda i: (0, i)),
        out_shape=jax.ShapeDtypeStruct((2 * N_EXP, n), jnp.float32),
        compiler_params=pltpu.CompilerParams(
            dimension_semantics=("arbitrary",),
        ),
    )(wc, x, bias.reshape(2 * N_EXP, 1))

    route = pl.kernel(
        _route_kernel,
        out_type=jax.ShapeDtypeStruct((n,), jnp.float32),
        mesh=plsc.VectorSubcoreMesh(core_axis_name="c", subcore_axis_name="s"),
        scratch_types=[
            pltpu.VMEM((2 * N_EXP, CHUNK), jnp.float32),
            pltpu.VMEM((CHUNK,), jnp.float32),
        ],
    )
    return route(comb_t).reshape(n, 1)
